# baseline (speedup 1.0000x reference)
import functools

import jax
import jax.numpy as jnp
from jax import lax
from jax.experimental import pallas as pl
from jax.experimental.pallas import tpu as pltpu
from jax.experimental.pallas import tpu_sc as plsc

N_NODES = 10000
IN_FEAT = 128
OUT_FEAT = 128

NPAD = 10240
NW = 32
ROWS_PER_TILE = NPAD // 16
CHUNK = 128


def _sc_body(feat_hbm, src_hbm, dst_hbm, seg_out, cnt_out,
             src_v, d_cur, rows_v, ones_v, zc_v,
             seg_sh, cnt_sh, gsem,
             *, chunks_per_tile):
    cid = lax.axis_index("c")
    sid = lax.axis_index("s")

    cpt0, cpt1 = chunks_per_tile
    my_cpt = jnp.where(cid == 0, cpt0, cpt1)
    base_c = jnp.where(cid == 0, sid * cpt0, 16 * cpt0 + sid * cpt1)
    base_e = base_c * CHUNK

    zrow = jnp.zeros((16,), jnp.float32)

    def zero_rows(i, _):
        for j in range(IN_FEAT // 16):
            rows_v[i, pl.ds(j * 16, 16)] = zrow
        return 0
    lax.fori_loop(0, CHUNK, zero_rows, 0)

    def zero_zc(i, _):
        zc_v[pl.ds(i * 16, 16)] = zrow
        return 0
    lax.fori_loop(0, ROWS_PER_TILE // 16, zero_zc, 0)

    for j in range(CHUNK // 16):
        ones_v[pl.ds(j * 16, 16)] = jnp.ones((16,), jnp.float32)

    base_n = sid * ROWS_PER_TILE
    for t in range(ROWS_PER_TILE // CHUNK):
        pltpu.sync_copy(rows_v, seg_sh.at[pl.ds(base_n + t * CHUNK, CHUNK)])
    pltpu.sync_copy(zc_v, cnt_sh.at[pl.ds(base_n, ROWS_PER_TILE)])

    plsc.subcore_barrier()

    def edge_step(t, _):
        off = base_e + t * CHUNK
        pltpu.sync_copy(src_hbm.at[pl.ds(off, CHUNK)], src_v)
        pltpu.sync_copy(dst_hbm.at[pl.ds(off, CHUNK)], d_cur)
        pltpu.async_copy(feat_hbm.at[src_v], rows_v, gsem).wait()
        pltpu.sync_copy(rows_v, seg_sh.at[d_cur], add=True)
        pltpu.sync_copy(ones_v, cnt_sh.at[d_cur], add=True)
        return 0

    lax.fori_loop(0, my_cpt, edge_step, 0)

    plsc.subcore_barrier()

    pltpu.sync_copy(seg_sh.at[pl.ds(base_n, ROWS_PER_TILE)],
                    seg_out.at[cid, pl.ds(base_n, ROWS_PER_TILE)])
    pltpu.sync_copy(cnt_sh.at[pl.ds(base_n, ROWS_PER_TILE)],
                    cnt_out.at[cid, pl.ds(base_n, ROWS_PER_TILE)])


def _segment_sum_sc(features, src, dst, chunks_per_tile):
    mesh = plsc.VectorSubcoreMesh(core_axis_name="c", subcore_axis_name="s")
    body = functools.partial(_sc_body, chunks_per_tile=chunks_per_tile)
    cpt_max = max(chunks_per_tile)
    return pl.kernel(
        body,
        out_type=[
            jax.ShapeDtypeStruct((2, NPAD, IN_FEAT), jnp.float32),
            jax.ShapeDtypeStruct((2, NPAD), jnp.float32),
        ],
        mesh=mesh,
        scratch_types=[
            pltpu.VMEM((CHUNK,), jnp.int32),
            pltpu.VMEM((CHUNK,), jnp.int32),
            pltpu.VMEM((CHUNK, IN_FEAT), jnp.float32),
            pltpu.VMEM((CHUNK,), jnp.float32),
            pltpu.VMEM((ROWS_PER_TILE,), jnp.float32),
            pltpu.VMEM_SHARED((NPAD, IN_FEAT), jnp.float32),
            pltpu.VMEM_SHARED((NPAD,), jnp.float32),
            pltpu.SemaphoreType.DMA,
        ],
    )(features, src, dst)


def _tc_body(feat_ref, w_ref, seg_ref, cnt_ref, out_ref):
    i = pl.program_id(0)
    blk = feat_ref.shape[0]
    w = w_ref[...]
    nodes_rep = jnp.dot(feat_ref[...], w, preferred_element_type=jnp.float32)
    seg = seg_ref[0] + seg_ref[1]
    cnt = cnt_ref[0, pl.ds(i * blk, blk)] + cnt_ref[1, pl.ds(i * blk, blk)]
    agg = seg / jnp.maximum(cnt, 1.0)[:, None]
    msgs = jnp.dot(agg, w, preferred_element_type=jnp.float32)
    out_ref[:, :OUT_FEAT] = jnp.maximum(nodes_rep, 0.0)
    out_ref[:, OUT_FEAT:] = jnp.maximum(msgs, 0.0)


def _combine_tc(feat_pad, W, seg_p, cnt_p):
    blk = 1024
    grid = (NPAD // blk,)
    return pl.pallas_call(
        _tc_body,
        grid=grid,
        in_specs=[
            pl.BlockSpec((blk, IN_FEAT), lambda i: (i, 0)),
            pl.BlockSpec((IN_FEAT, OUT_FEAT), lambda i: (0, 0)),
            pl.BlockSpec((2, blk, IN_FEAT), lambda i: (0, i, 0)),
            pl.BlockSpec((2, NPAD), lambda i: (0, 0)),
        ],
        out_specs=pl.BlockSpec((blk, 2 * OUT_FEAT), lambda i: (i, 0)),
        out_shape=jax.ShapeDtypeStruct((NPAD, 2 * OUT_FEAT), jnp.float32),
    )(feat_pad, W, seg_p, cnt_p)


CORE0_SHARE = 0.5


def kernel(features, edge_index, W):
    n_edges = edge_index.shape[1]
    cpt = -(-n_edges // (NW * CHUNK))
    cpt += cpt % 2
    total_chunks = cpt * NW
    cpt0 = max(2, int(round(total_chunks * CORE0_SHARE / 16 / 2)) * 2)
    cpt1 = total_chunks // 16 - cpt0
    epad = total_chunks * CHUNK
    ei = edge_index.astype(jnp.int32)
    pad = epad - n_edges
    src = jnp.concatenate([ei[1], jnp.zeros((pad,), jnp.int32)])
    dst = jnp.concatenate([ei[0], jnp.full((pad,), N_NODES, jnp.int32)])

    seg_p, cnt_p = _segment_sum_sc(features, src, dst, (cpt0, cpt1))

    feat_pad = jnp.pad(features, ((0, NPAD - N_NODES), (0, 0)))
    out = _combine_tc(feat_pad, W, seg_p, cnt_p)
    return out[:N_NODES]


# speedup vs baseline: 1.0001x; 1.0001x over previous
import functools

import jax
import jax.numpy as jnp
from jax import lax
from jax.experimental import pallas as pl
from jax.experimental.pallas import tpu as pltpu
from jax.experimental.pallas import tpu_sc as plsc

N_NODES = 10000
IN_FEAT = 128
OUT_FEAT = 128

NPAD = 10240
NW = 32
ROWS_PER_TILE = NPAD // 16
CHUNK = 128


def _sc_body(feat_hbm, src_hbm, dst_hbm, seg_out, cnt_out,
             src_v, d_cur, rows_v, ones_v, zc_v,
             seg_sh, cnt_sh, gsem,
             *, chunks_per_tile):
    cid = lax.axis_index("c")
    sid = lax.axis_index("s")

    cpt0, cpt1 = chunks_per_tile
    my_cpt = jnp.where(cid == 0, cpt0, cpt1)
    base_c = jnp.where(cid == 0, sid * cpt0, 16 * cpt0 + sid * cpt1)
    base_e = base_c * CHUNK

    zrow = jnp.zeros((16,), jnp.float32)

    def zero_rows(i, _):
        for j in range(IN_FEAT // 16):
            rows_v[i, pl.ds(j * 16, 16)] = zrow
        return 0
    lax.fori_loop(0, CHUNK, zero_rows, 0)

    def zero_zc(i, _):
        zc_v[pl.ds(i * 16, 16)] = zrow
        return 0
    lax.fori_loop(0, ROWS_PER_TILE // 16, zero_zc, 0)

    for j in range(CHUNK // 16):
        ones_v[pl.ds(j * 16, 16)] = jnp.ones((16,), jnp.float32)

    base_n = sid * ROWS_PER_TILE
    for t in range(ROWS_PER_TILE // CHUNK):
        pltpu.sync_copy(rows_v, seg_sh.at[pl.ds(base_n + t * CHUNK, CHUNK)])
    pltpu.sync_copy(zc_v, cnt_sh.at[pl.ds(base_n, ROWS_PER_TILE)])

    plsc.subcore_barrier()

    def edge_step(t, _):
        off = base_e + t * CHUNK
        pltpu.sync_copy(src_hbm.at[pl.ds(off, CHUNK)], src_v)
        pltpu.sync_copy(dst_hbm.at[pl.ds(off, CHUNK)], d_cur)
        pltpu.async_copy(feat_hbm.at[src_v], rows_v, gsem).wait()
        pltpu.sync_copy(rows_v, seg_sh.at[d_cur], add=True)
        pltpu.sync_copy(ones_v, cnt_sh.at[d_cur], add=True)
        return 0

    lax.fori_loop(0, my_cpt, edge_step, 0)

    plsc.subcore_barrier()

    pltpu.sync_copy(seg_sh.at[pl.ds(base_n, ROWS_PER_TILE)],
                    seg_out.at[cid, pl.ds(base_n, ROWS_PER_TILE)])
    pltpu.sync_copy(cnt_sh.at[pl.ds(base_n, ROWS_PER_TILE)],
                    cnt_out.at[cid, pl.ds(base_n, ROWS_PER_TILE)])


def _segment_sum_sc(features, src, dst, chunks_per_tile):
    mesh = plsc.VectorSubcoreMesh(core_axis_name="c", subcore_axis_name="s")
    body = functools.partial(_sc_body, chunks_per_tile=chunks_per_tile)
    cpt_max = max(chunks_per_tile)
    return pl.kernel(
        body,
        out_type=[
            jax.ShapeDtypeStruct((2, NPAD, IN_FEAT), jnp.float32),
            jax.ShapeDtypeStruct((2, NPAD), jnp.float32),
        ],
        mesh=mesh,
        scratch_types=[
            pltpu.VMEM((CHUNK,), jnp.int32),
            pltpu.VMEM((CHUNK,), jnp.int32),
            pltpu.VMEM((CHUNK, IN_FEAT), jnp.float32),
            pltpu.VMEM((CHUNK,), jnp.float32),
            pltpu.VMEM((ROWS_PER_TILE,), jnp.float32),
            pltpu.VMEM_SHARED((NPAD, IN_FEAT), jnp.float32),
            pltpu.VMEM_SHARED((NPAD,), jnp.float32),
            pltpu.SemaphoreType.DMA,
        ],
    )(features, src, dst)


def _tc_body(feat_ref, w_ref, seg_ref, cnt_ref, out_ref):
    i = pl.program_id(0)
    blk = feat_ref.shape[0]
    w = w_ref[...]
    nodes_rep = jnp.dot(feat_ref[...], w, preferred_element_type=jnp.float32)
    seg = seg_ref[0] + seg_ref[1]
    cnt = cnt_ref[0, pl.ds(i * blk, blk)] + cnt_ref[1, pl.ds(i * blk, blk)]
    agg = seg / jnp.maximum(cnt, 1.0)[:, None]
    msgs = jnp.dot(agg, w, preferred_element_type=jnp.float32)
    out_ref[:, :OUT_FEAT] = jnp.maximum(nodes_rep, 0.0)
    out_ref[:, OUT_FEAT:] = jnp.maximum(msgs, 0.0)


def _combine_tc(feat_pad, W, seg_p, cnt_p):
    blk = 1024
    grid = (NPAD // blk,)
    return pl.pallas_call(
        _tc_body,
        grid=grid,
        in_specs=[
            pl.BlockSpec((blk, IN_FEAT), lambda i: (i, 0)),
            pl.BlockSpec((IN_FEAT, OUT_FEAT), lambda i: (0, 0)),
            pl.BlockSpec((2, blk, IN_FEAT), lambda i: (0, i, 0)),
            pl.BlockSpec((2, NPAD), lambda i: (0, 0)),
        ],
        out_specs=pl.BlockSpec((blk, 2 * OUT_FEAT), lambda i: (i, 0)),
        out_shape=jax.ShapeDtypeStruct((NPAD, 2 * OUT_FEAT), jnp.float32),
    )(feat_pad, W, seg_p, cnt_p)


CORE0_SHARE = 0.5


def kernel(features, edge_index, W):
    n_edges = edge_index.shape[1]
    cpt = -(-n_edges // (NW * CHUNK))
    cpt += cpt % 2
    total_chunks = cpt * NW
    cpt0 = max(2, int(round(total_chunks * CORE0_SHARE / 16 / 2)) * 2)
    cpt1 = total_chunks // 16 - cpt0
    epad = total_chunks * CHUNK
    ei = edge_index.astype(jnp.int32)
    pad = epad - n_edges
    src = jnp.concatenate([ei[1], jnp.zeros((pad,), jnp.int32)])
    dummy = N_NODES + jnp.arange(pad, dtype=jnp.int32) % (NPAD - N_NODES)
    dst = jnp.concatenate([ei[0], dummy])

    seg_p, cnt_p = _segment_sum_sc(features, src, dst, (cpt0, cpt1))

    feat_pad = jnp.pad(features, ((0, NPAD - N_NODES), (0, 0)))
    out = _combine_tc(feat_pad, W, seg_p, cnt_p)
    return out[:N_NODES]


# speedup vs baseline: 1.0016x; 1.0015x over previous
import functools

import jax
import jax.numpy as jnp
from jax import lax
from jax.experimental import pallas as pl
from jax.experimental.pallas import tpu as pltpu
from jax.experimental.pallas import tpu_sc as plsc

N_NODES = 10000
IN_FEAT = 128
OUT_FEAT = 128

NPAD = 10240
NW = 32
ROWS_PER_TILE = NPAD // 16
CHUNK = 128


def _sc_body(feat_hbm, src_hbm, dst_hbm, seg_out, cnt_out,
             src_v, d_cur, rows_v, ones_v, zc_v,
             seg_sh, cnt_sh, gsem,
             *, chunks_per_tile):
    cid = lax.axis_index("c")
    sid = lax.axis_index("s")

    cpt0, cpt1 = chunks_per_tile
    my_cpt = cpt0
    wid = sid * 2 + cid
    base_e = wid * (cpt0 * CHUNK)

    zrow = jnp.zeros((16,), jnp.float32)

    def zero_rows(i, _):
        for j in range(IN_FEAT // 16):
            rows_v[i, pl.ds(j * 16, 16)] = zrow
        return 0
    lax.fori_loop(0, CHUNK, zero_rows, 0)

    def zero_zc(i, _):
        zc_v[pl.ds(i * 16, 16)] = zrow
        return 0
    lax.fori_loop(0, ROWS_PER_TILE // 16, zero_zc, 0)

    for j in range(CHUNK // 16):
        ones_v[pl.ds(j * 16, 16)] = jnp.ones((16,), jnp.float32)

    base_n = sid * ROWS_PER_TILE
    for t in range(ROWS_PER_TILE // CHUNK):
        pltpu.sync_copy(rows_v, seg_sh.at[pl.ds(base_n + t * CHUNK, CHUNK)])
    pltpu.sync_copy(zc_v, cnt_sh.at[pl.ds(base_n, ROWS_PER_TILE)])

    plsc.subcore_barrier()

    def edge_step(t, _):
        off = base_e + t * CHUNK
        pltpu.sync_copy(src_hbm.at[pl.ds(off, CHUNK)], src_v)
        pltpu.sync_copy(dst_hbm.at[pl.ds(off, CHUNK)], d_cur)
        pltpu.async_copy(feat_hbm.at[src_v], rows_v, gsem).wait()
        pltpu.sync_copy(rows_v, seg_sh.at[d_cur], add=True)
        pltpu.sync_copy(ones_v, cnt_sh.at[d_cur], add=True)
        return 0

    lax.fori_loop(0, my_cpt, edge_step, 0)

    plsc.subcore_barrier()

    pltpu.sync_copy(seg_sh.at[pl.ds(base_n, ROWS_PER_TILE)],
                    seg_out.at[cid, pl.ds(base_n, ROWS_PER_TILE)])
    pltpu.sync_copy(cnt_sh.at[pl.ds(base_n, ROWS_PER_TILE)],
                    cnt_out.at[cid, pl.ds(base_n, ROWS_PER_TILE)])


def _segment_sum_sc(features, src, dst, chunks_per_tile):
    mesh = plsc.VectorSubcoreMesh(core_axis_name="c", subcore_axis_name="s")
    body = functools.partial(_sc_body, chunks_per_tile=chunks_per_tile)
    cpt_max = max(chunks_per_tile)
    return pl.kernel(
        body,
        out_type=[
            jax.ShapeDtypeStruct((2, NPAD, IN_FEAT), jnp.float32),
            jax.ShapeDtypeStruct((2, NPAD), jnp.float32),
        ],
        mesh=mesh,
        scratch_types=[
            pltpu.VMEM((CHUNK,), jnp.int32),
            pltpu.VMEM((CHUNK,), jnp.int32),
            pltpu.VMEM((CHUNK, IN_FEAT), jnp.float32),
            pltpu.VMEM((CHUNK,), jnp.float32),
            pltpu.VMEM((ROWS_PER_TILE,), jnp.float32),
            pltpu.VMEM_SHARED((NPAD, IN_FEAT), jnp.float32),
            pltpu.VMEM_SHARED((NPAD,), jnp.float32),
            pltpu.SemaphoreType.DMA,
        ],
    )(features, src, dst)


def _tc_body(feat_ref, w_ref, seg_ref, cnt_ref, out_ref):
    i = pl.program_id(0)
    blk = feat_ref.shape[0]
    w = w_ref[...]
    nodes_rep = jnp.dot(feat_ref[...], w, preferred_element_type=jnp.float32)
    seg = seg_ref[0] + seg_ref[1]
    cnt = cnt_ref[0, pl.ds(i * blk, blk)] + cnt_ref[1, pl.ds(i * blk, blk)]
    agg = seg / jnp.maximum(cnt, 1.0)[:, None]
    msgs = jnp.dot(agg, w, preferred_element_type=jnp.float32)
    out_ref[:, :OUT_FEAT] = jnp.maximum(nodes_rep, 0.0)
    out_ref[:, OUT_FEAT:] = jnp.maximum(msgs, 0.0)


def _combine_tc(feat_pad, W, seg_p, cnt_p):
    blk = 1024
    grid = (NPAD // blk,)
    return pl.pallas_call(
        _tc_body,
        grid=grid,
        in_specs=[
            pl.BlockSpec((blk, IN_FEAT), lambda i: (i, 0)),
            pl.BlockSpec((IN_FEAT, OUT_FEAT), lambda i: (0, 0)),
            pl.BlockSpec((2, blk, IN_FEAT), lambda i: (0, i, 0)),
            pl.BlockSpec((2, NPAD), lambda i: (0, 0)),
        ],
        out_specs=pl.BlockSpec((blk, 2 * OUT_FEAT), lambda i: (i, 0)),
        out_shape=jax.ShapeDtypeStruct((NPAD, 2 * OUT_FEAT), jnp.float32),
    )(feat_pad, W, seg_p, cnt_p)


CORE0_SHARE = 0.5


def kernel(features, edge_index, W):
    n_edges = edge_index.shape[1]
    cpt = -(-n_edges // (NW * CHUNK))
    cpt += cpt % 2
    total_chunks = cpt * NW
    cpt0 = max(2, int(round(total_chunks * CORE0_SHARE / 16 / 2)) * 2)
    cpt1 = total_chunks // 16 - cpt0
    epad = total_chunks * CHUNK
    ei = edge_index.astype(jnp.int32)
    pad = epad - n_edges
    src = jnp.concatenate([ei[1], jnp.zeros((pad,), jnp.int32)])
    dummy = N_NODES + jnp.arange(pad, dtype=jnp.int32) % (NPAD - N_NODES)
    dst = jnp.concatenate([ei[0], dummy])

    seg_p, cnt_p = _segment_sum_sc(features, src, dst, (cpt0, cpt1))

    feat_pad = jnp.pad(features, ((0, NPAD - N_NODES), (0, 0)))
    out = _combine_tc(feat_pad, W, seg_p, cnt_p)
    return out[:N_NODES]


# speedup vs baseline: 2.2465x; 2.2430x over previous
import functools

import jax
import jax.numpy as jnp
from jax import lax
from jax.experimental import pallas as pl
from jax.experimental.pallas import tpu as pltpu
from jax.experimental.pallas import tpu_sc as plsc

N_NODES = 10000
IN_FEAT = 128
OUT_FEAT = 128

NPAD = 10240
NW = 32
ROWS_PER_TILE = NPAD // 16
CHUNK = 128


def _sc_body(feat_hbm, src_hbm, dst_hbm, seg_out, cnt_out,
             src_v, d_cur, rows_v, ones_v, zc_v,
             seg_sh, cnt_sh, gsem,
             *, chunks_per_tile):
    cid = lax.axis_index("c")
    sid = lax.axis_index("s")

    cpt0, cpt1 = chunks_per_tile
    my_cpt = cpt0
    wid = sid * 2 + cid
    base_e = wid * (cpt0 * CHUNK)

    zrow = jnp.zeros((16,), jnp.float32)

    def zero_rows(i, _):
        for j in range(IN_FEAT // 16):
            rows_v[i, pl.ds(j * 16, 16)] = zrow
        return 0
    lax.fori_loop(0, CHUNK, zero_rows, 0)

    def zero_zc(i, _):
        zc_v[pl.ds(i * 16, 16)] = zrow
        return 0
    lax.fori_loop(0, ROWS_PER_TILE // 16, zero_zc, 0)

    for j in range(CHUNK // 16):
        ones_v[pl.ds(j * 16, 16)] = jnp.ones((16,), jnp.float32)

    base_n = sid * ROWS_PER_TILE
    for t in range(ROWS_PER_TILE // CHUNK):
        pltpu.sync_copy(rows_v, seg_sh.at[pl.ds(base_n + t * CHUNK, CHUNK)])
    pltpu.sync_copy(zc_v, cnt_sh.at[pl.ds(base_n, ROWS_PER_TILE)])

    plsc.subcore_barrier()

    def edge_step(t, _):
        off = base_e + t * CHUNK
        pltpu.sync_copy(src_hbm.at[pl.ds(off, CHUNK)], src_v)
        pltpu.sync_copy(dst_hbm.at[pl.ds(off, CHUNK)], d_cur)
        pltpu.async_copy(feat_hbm.at[src_v], rows_v, gsem).wait()
        pltpu.sync_copy(rows_v, seg_sh.at[d_cur], add=True)
        pltpu.sync_copy(ones_v, cnt_sh.at[d_cur], add=True)
        return 0

    lax.fori_loop(0, my_cpt, edge_step, 0)

    plsc.subcore_barrier()

    pltpu.sync_copy(seg_sh.at[pl.ds(base_n, ROWS_PER_TILE)],
                    seg_out.at[cid, pl.ds(base_n, ROWS_PER_TILE)])
    pltpu.sync_copy(cnt_sh.at[pl.ds(base_n, ROWS_PER_TILE)],
                    cnt_out.at[cid, pl.ds(base_n, ROWS_PER_TILE)])


def _segment_sum_sc(features, src, dst, chunks_per_tile):
    mesh = plsc.VectorSubcoreMesh(core_axis_name="c", subcore_axis_name="s")
    body = functools.partial(_sc_body, chunks_per_tile=chunks_per_tile)
    cpt_max = max(chunks_per_tile)
    return pl.kernel(
        body,
        out_type=[
            jax.ShapeDtypeStruct((2, NPAD, IN_FEAT), jnp.float32),
            jax.ShapeDtypeStruct((2, NPAD), jnp.float32),
        ],
        mesh=mesh,
        scratch_types=[
            pltpu.VMEM((CHUNK,), jnp.int32),
            pltpu.VMEM((CHUNK,), jnp.int32),
            pltpu.VMEM((CHUNK, IN_FEAT), jnp.float32),
            pltpu.VMEM((CHUNK,), jnp.float32),
            pltpu.VMEM((ROWS_PER_TILE,), jnp.float32),
            pltpu.VMEM_SHARED((NPAD, IN_FEAT), jnp.float32),
            pltpu.VMEM_SHARED((NPAD,), jnp.float32),
            pltpu.SemaphoreType.DMA,
        ],
    )(features, src, dst)


def _tc_body(feat_ref, w_ref, seg_ref, cnt_ref, out_ref):
    i = pl.program_id(0)
    blk = feat_ref.shape[0]
    w = w_ref[...]
    nodes_rep = jnp.dot(feat_ref[...], w, preferred_element_type=jnp.float32)
    seg = seg_ref[0] + seg_ref[1]
    cnt = cnt_ref[0, pl.ds(i * blk, blk)] + cnt_ref[1, pl.ds(i * blk, blk)]
    agg = seg / jnp.maximum(cnt, 1.0)[:, None]
    msgs = jnp.dot(agg, w, preferred_element_type=jnp.float32)
    out_ref[:, :OUT_FEAT] = jnp.maximum(nodes_rep, 0.0)
    out_ref[:, OUT_FEAT:] = jnp.maximum(msgs, 0.0)


def _combine_tc(feat_pad, W, seg_p, cnt_p):
    blk = 1024
    grid = (NPAD // blk,)
    return pl.pallas_call(
        _tc_body,
        grid=grid,
        in_specs=[
            pl.BlockSpec((blk, IN_FEAT), lambda i: (i, 0)),
            pl.BlockSpec((IN_FEAT, OUT_FEAT), lambda i: (0, 0)),
            pl.BlockSpec((2, blk, IN_FEAT), lambda i: (0, i, 0)),
            pl.BlockSpec((2, NPAD), lambda i: (0, 0)),
        ],
        out_specs=pl.BlockSpec((blk, 2 * OUT_FEAT), lambda i: (i, 0)),
        out_shape=jax.ShapeDtypeStruct((NPAD, 2 * OUT_FEAT), jnp.float32),
    )(feat_pad, W, seg_p, cnt_p)


CORE0_SHARE = 0.5


def kernel(features, edge_index, W):
    n_edges = edge_index.shape[1]
    cpt = -(-n_edges // (NW * CHUNK))
    cpt += cpt % 2
    total_chunks = cpt * NW
    cpt0 = max(2, int(round(total_chunks * CORE0_SHARE / 16 / 2)) * 2)
    cpt1 = total_chunks // 16 - cpt0
    epad = total_chunks * CHUNK
    ei = edge_index.astype(jnp.int32)
    pad = epad - n_edges
    pad_ar = jnp.arange(pad, dtype=jnp.int32)
    src = jnp.concatenate([ei[1], (pad_ar * 37) % N_NODES])
    dst = jnp.concatenate([ei[0], N_NODES + pad_ar % (NPAD - N_NODES)])

    seg_p, cnt_p = _segment_sum_sc(features, src, dst, (cpt0, cpt1))

    feat_pad = jnp.pad(features, ((0, NPAD - N_NODES), (0, 0)))
    out = _combine_tc(feat_pad, W, seg_p, cnt_p)
    return out[:N_NODES]


# speedup vs baseline: 3.3056x; 1.4714x over previous
import functools

import jax
import jax.numpy as jnp
from jax import lax
from jax.experimental import pallas as pl
from jax.experimental.pallas import tpu as pltpu
from jax.experimental.pallas import tpu_sc as plsc

N_NODES = 10000
IN_FEAT = 128
OUT_FEAT = 128

NPAD = 10240
NW = 32
ROWS_PER_TILE = NPAD // 16
CHUNK = 128


def _sc_body(feat_hbm, src_hbm, dst_hbm, seg_out, cnt_out,
             src_a, src_b, dst_a, dst_b, rows_a, rows_b, ones_v, zc_v,
             seg_sh, cnt_sh, gsem_a, gsem_b,
             *, chunks_per_tile):
    cid = lax.axis_index("c")
    sid = lax.axis_index("s")

    cpt0, cpt1 = chunks_per_tile
    my_cpt = cpt0
    wid = sid * 2 + cid
    base_e = wid * (cpt0 * CHUNK)

    zrow = jnp.zeros((16,), jnp.float32)

    def zero_rows(i, _):
        for j in range(IN_FEAT // 16):
            rows_a[i, pl.ds(j * 16, 16)] = zrow
        return 0
    lax.fori_loop(0, CHUNK, zero_rows, 0)

    def zero_zc(i, _):
        zc_v[pl.ds(i * 16, 16)] = zrow
        return 0
    lax.fori_loop(0, ROWS_PER_TILE // 16, zero_zc, 0)

    for j in range(CHUNK // 16):
        ones_v[pl.ds(j * 16, 16)] = jnp.ones((16,), jnp.float32)

    base_n = sid * ROWS_PER_TILE
    for t in range(ROWS_PER_TILE // CHUNK):
        pltpu.sync_copy(rows_a, seg_sh.at[pl.ds(base_n + t * CHUNK, CHUNK)])
    pltpu.sync_copy(zc_v, cnt_sh.at[pl.ds(base_n, ROWS_PER_TILE)])

    plsc.subcore_barrier()

    last = my_cpt - 1

    def idx_load_sync(t, s_v, d_v):
        off = base_e + jnp.minimum(t, last) * CHUNK
        pltpu.sync_copy(src_hbm.at[pl.ds(off, CHUNK)], s_v)
        pltpu.sync_copy(dst_hbm.at[pl.ds(off, CHUNK)], d_v)

    def process(r_v, d_v):
        pltpu.sync_copy(r_v, seg_sh.at[d_v], add=True)
        pltpu.sync_copy(ones_v, cnt_sh.at[d_v], add=True)

    idx_load_sync(jnp.int32(0), src_a, dst_a)
    idx_load_sync(jnp.int32(1), src_b, dst_b)
    pltpu.async_copy(feat_hbm.at[src_a], rows_a, gsem_a).wait()

    def edge_pair(i, _):
        t = 2 * i
        cb = pltpu.async_copy(feat_hbm.at[src_b], rows_b, gsem_b)
        process(rows_a, dst_a)
        idx_load_sync(t + 2, src_a, dst_a)
        cb.wait()
        ca = pltpu.async_copy(feat_hbm.at[src_a], rows_a, gsem_a)
        process(rows_b, dst_b)
        idx_load_sync(t + 3, src_b, dst_b)
        ca.wait()
        return 0
    lax.fori_loop(0, my_cpt // 2, edge_pair, 0)

    plsc.subcore_barrier()

    pltpu.sync_copy(seg_sh.at[pl.ds(base_n, ROWS_PER_TILE)],
                    seg_out.at[cid, pl.ds(base_n, ROWS_PER_TILE)])
    pltpu.sync_copy(cnt_sh.at[pl.ds(base_n, ROWS_PER_TILE)],
                    cnt_out.at[cid, pl.ds(base_n, ROWS_PER_TILE)])


def _segment_sum_sc(features, src, dst, chunks_per_tile):
    mesh = plsc.VectorSubcoreMesh(core_axis_name="c", subcore_axis_name="s")
    body = functools.partial(_sc_body, chunks_per_tile=chunks_per_tile)
    cpt_max = max(chunks_per_tile)
    return pl.kernel(
        body,
        out_type=[
            jax.ShapeDtypeStruct((2, NPAD, IN_FEAT), jnp.float32),
            jax.ShapeDtypeStruct((2, NPAD), jnp.float32),
        ],
        mesh=mesh,
        scratch_types=[
            pltpu.VMEM((CHUNK,), jnp.int32),
            pltpu.VMEM((CHUNK,), jnp.int32),
            pltpu.VMEM((CHUNK,), jnp.int32),
            pltpu.VMEM((CHUNK,), jnp.int32),
            pltpu.VMEM((CHUNK, IN_FEAT), jnp.float32),
            pltpu.VMEM((CHUNK, IN_FEAT), jnp.float32),
            pltpu.VMEM((CHUNK,), jnp.float32),
            pltpu.VMEM((ROWS_PER_TILE,), jnp.float32),
            pltpu.VMEM_SHARED((NPAD, IN_FEAT), jnp.float32),
            pltpu.VMEM_SHARED((NPAD,), jnp.float32),
            pltpu.SemaphoreType.DMA,
            pltpu.SemaphoreType.DMA,
        ],
    )(features, src, dst)


def _tc_body(feat_ref, w_ref, seg_ref, cnt_ref, out_ref):
    i = pl.program_id(0)
    blk = feat_ref.shape[0]
    w = w_ref[...]
    nodes_rep = jnp.dot(feat_ref[...], w, preferred_element_type=jnp.float32)
    seg = seg_ref[0] + seg_ref[1]
    cnt = cnt_ref[0, pl.ds(i * blk, blk)] + cnt_ref[1, pl.ds(i * blk, blk)]
    agg = seg / jnp.maximum(cnt, 1.0)[:, None]
    msgs = jnp.dot(agg, w, preferred_element_type=jnp.float32)
    out_ref[:, :OUT_FEAT] = jnp.maximum(nodes_rep, 0.0)
    out_ref[:, OUT_FEAT:] = jnp.maximum(msgs, 0.0)


def _combine_tc(feat_pad, W, seg_p, cnt_p):
    blk = 1024
    grid = (NPAD // blk,)
    return pl.pallas_call(
        _tc_body,
        grid=grid,
        in_specs=[
            pl.BlockSpec((blk, IN_FEAT), lambda i: (i, 0)),
            pl.BlockSpec((IN_FEAT, OUT_FEAT), lambda i: (0, 0)),
            pl.BlockSpec((2, blk, IN_FEAT), lambda i: (0, i, 0)),
            pl.BlockSpec((2, NPAD), lambda i: (0, 0)),
        ],
        out_specs=pl.BlockSpec((blk, 2 * OUT_FEAT), lambda i: (i, 0)),
        out_shape=jax.ShapeDtypeStruct((NPAD, 2 * OUT_FEAT), jnp.float32),
    )(feat_pad, W, seg_p, cnt_p)


CORE0_SHARE = 0.5


def kernel(features, edge_index, W):
    n_edges = edge_index.shape[1]
    cpt = -(-n_edges // (NW * CHUNK))
    cpt += cpt % 2
    total_chunks = cpt * NW
    cpt0 = max(2, int(round(total_chunks * CORE0_SHARE / 16 / 2)) * 2)
    cpt1 = total_chunks // 16 - cpt0
    epad = total_chunks * CHUNK
    ei = edge_index.astype(jnp.int32)
    pad = epad - n_edges
    pad_ar = jnp.arange(pad, dtype=jnp.int32)
    src = jnp.concatenate([ei[1], (pad_ar * 37) % N_NODES])
    dst = jnp.concatenate([ei[0], N_NODES + pad_ar % (NPAD - N_NODES)])

    seg_p, cnt_p = _segment_sum_sc(features, src, dst, (cpt0, cpt1))

    feat_pad = jnp.pad(features, ((0, NPAD - N_NODES), (0, 0)))
    out = _combine_tc(feat_pad, W, seg_p, cnt_p)
    return out[:N_NODES]


# speedup vs baseline: 3.6618x; 1.1078x over previous
import functools

import jax
import jax.numpy as jnp
from jax import lax
from jax.experimental import pallas as pl
from jax.experimental.pallas import tpu as pltpu
from jax.experimental.pallas import tpu_sc as plsc

N_NODES = 10000
IN_FEAT = 128
OUT_FEAT = 128

NPAD = 10240
NW = 32
ROWS_PER_TILE = NPAD // 16
CHUNK = 128
KBLK = 8


def _sc_body(feat_hbm, src_hbm, dst_hbm, seg_out, cnt_out,
             src_a, src_b, dst_a, dst_b, rows_a, rows_b, ones_v, zc_v,
             seg_sh, cnt_sh, gsem_a, gsem_b, isem,
             *, chunks_per_tile):
    cid = lax.axis_index("c")
    sid = lax.axis_index("s")

    cpt0, cpt1 = chunks_per_tile
    my_cpt = cpt0
    wid = sid * 2 + cid
    base_c = wid * cpt0

    zrow = jnp.zeros((16,), jnp.float32)

    def zero_rows(i, _):
        for j in range(IN_FEAT // 16):
            rows_a[i, pl.ds(j * 16, 16)] = zrow
        return 0
    lax.fori_loop(0, CHUNK, zero_rows, 0)

    def zero_zc(i, _):
        zc_v[pl.ds(i * 16, 16)] = zrow
        return 0
    lax.fori_loop(0, ROWS_PER_TILE // 16, zero_zc, 0)

    for j in range(CHUNK // 16):
        ones_v[pl.ds(j * 16, 16)] = jnp.ones((16,), jnp.float32)

    base_n = sid * ROWS_PER_TILE
    for t in range(ROWS_PER_TILE // CHUNK):
        pltpu.sync_copy(rows_a, seg_sh.at[pl.ds(base_n + t * CHUNK, CHUNK)])
    pltpu.sync_copy(zc_v, cnt_sh.at[pl.ds(base_n, ROWS_PER_TILE)])

    plsc.subcore_barrier()

    nblk = my_cpt // KBLK
    lastb = nblk - 1

    def blk_load(b, s_v, d_v):
        off = base_c + jnp.minimum(b, lastb) * KBLK
        pltpu.async_copy(src_hbm.at[pl.ds(off, KBLK)], s_v, isem)
        pltpu.async_copy(dst_hbm.at[pl.ds(off, KBLK)], d_v, isem)

    def blk_wait(s_v, d_v):
        pltpu.make_async_copy(src_hbm.at[pl.ds(0, KBLK)], s_v, isem).wait()
        pltpu.make_async_copy(dst_hbm.at[pl.ds(0, KBLK)], d_v, isem).wait()

    def process(r_v, d_v):
        pltpu.sync_copy(r_v, seg_sh.at[d_v], add=True)
        pltpu.sync_copy(ones_v, cnt_sh.at[d_v], add=True)

    def do_block(sblk, dblk):
        pltpu.async_copy(feat_hbm.at[sblk.at[0]], rows_a, gsem_a).wait()
        for k in range(KBLK // 2):
            cb = pltpu.async_copy(feat_hbm.at[sblk.at[2 * k + 1]], rows_b, gsem_b)
            process(rows_a, dblk.at[2 * k])
            cb.wait()
            if 2 * k + 2 < KBLK:
                ca = pltpu.async_copy(feat_hbm.at[sblk.at[2 * k + 2]], rows_a, gsem_a)
                process(rows_b, dblk.at[2 * k + 1])
                ca.wait()
            else:
                process(rows_b, dblk.at[2 * k + 1])

    blk_load(jnp.int32(0), src_a, dst_a)
    blk_wait(src_a, dst_a)
    blk_load(jnp.int32(1), src_b, dst_b)

    def block_pair(j, _):
        b = 2 * j
        do_block(src_a, dst_a)
        blk_wait(src_b, dst_b)
        blk_load(b + 2, src_a, dst_a)
        do_block(src_b, dst_b)
        blk_wait(src_a, dst_a)
        blk_load(b + 3, src_b, dst_b)
        return 0
    lax.fori_loop(0, nblk // 2, block_pair, 0)

    blk_wait(src_b, dst_b)

    plsc.subcore_barrier()

    pltpu.sync_copy(seg_sh.at[pl.ds(base_n, ROWS_PER_TILE)],
                    seg_out.at[cid, pl.ds(base_n, ROWS_PER_TILE)])
    pltpu.sync_copy(cnt_sh.at[pl.ds(base_n, ROWS_PER_TILE)],
                    cnt_out.at[cid, pl.ds(base_n, ROWS_PER_TILE)])


def _segment_sum_sc(features, src, dst, chunks_per_tile):
    mesh = plsc.VectorSubcoreMesh(core_axis_name="c", subcore_axis_name="s")
    body = functools.partial(_sc_body, chunks_per_tile=chunks_per_tile)
    cpt_max = max(chunks_per_tile)
    return pl.kernel(
        body,
        out_type=[
            jax.ShapeDtypeStruct((2, NPAD, IN_FEAT), jnp.float32),
            jax.ShapeDtypeStruct((2, NPAD), jnp.float32),
        ],
        mesh=mesh,
        scratch_types=[
            pltpu.VMEM((KBLK, CHUNK), jnp.int32),
            pltpu.VMEM((KBLK, CHUNK), jnp.int32),
            pltpu.VMEM((KBLK, CHUNK), jnp.int32),
            pltpu.VMEM((KBLK, CHUNK), jnp.int32),
            pltpu.VMEM((CHUNK, IN_FEAT), jnp.float32),
            pltpu.VMEM((CHUNK, IN_FEAT), jnp.float32),
            pltpu.VMEM((CHUNK,), jnp.float32),
            pltpu.VMEM((ROWS_PER_TILE,), jnp.float32),
            pltpu.VMEM_SHARED((NPAD, IN_FEAT), jnp.float32),
            pltpu.VMEM_SHARED((NPAD,), jnp.float32),
            pltpu.SemaphoreType.DMA,
            pltpu.SemaphoreType.DMA,
            pltpu.SemaphoreType.DMA,
        ],
    )(features, src, dst)


def _tc_body(feat_ref, w_ref, seg_ref, cnt_ref, out_ref):
    i = pl.program_id(0)
    blk = feat_ref.shape[0]
    w = w_ref[...]
    nodes_rep = jnp.dot(feat_ref[...], w, preferred_element_type=jnp.float32)
    seg = seg_ref[0] + seg_ref[1]
    cnt = cnt_ref[0, pl.ds(i * blk, blk)] + cnt_ref[1, pl.ds(i * blk, blk)]
    agg = seg / jnp.maximum(cnt, 1.0)[:, None]
    msgs = jnp.dot(agg, w, preferred_element_type=jnp.float32)
    out_ref[:, :OUT_FEAT] = jnp.maximum(nodes_rep, 0.0)
    out_ref[:, OUT_FEAT:] = jnp.maximum(msgs, 0.0)


def _combine_tc(feat_pad, W, seg_p, cnt_p):
    blk = 1024
    grid = (NPAD // blk,)
    return pl.pallas_call(
        _tc_body,
        grid=grid,
        in_specs=[
            pl.BlockSpec((blk, IN_FEAT), lambda i: (i, 0)),
            pl.BlockSpec((IN_FEAT, OUT_FEAT), lambda i: (0, 0)),
            pl.BlockSpec((2, blk, IN_FEAT), lambda i: (0, i, 0)),
            pl.BlockSpec((2, NPAD), lambda i: (0, 0)),
        ],
        out_specs=pl.BlockSpec((blk, 2 * OUT_FEAT), lambda i: (i, 0)),
        out_shape=jax.ShapeDtypeStruct((NPAD, 2 * OUT_FEAT), jnp.float32),
    )(feat_pad, W, seg_p, cnt_p)


def kernel(features, edge_index, W):
    n_edges = edge_index.shape[1]
    cpt = -(-n_edges // (NW * CHUNK))
    cpt = -(-cpt // (2 * KBLK)) * (2 * KBLK)
    total_chunks = cpt * NW
    cpt0 = cpt1 = cpt
    epad = total_chunks * CHUNK
    ei = edge_index.astype(jnp.int32)
    pad = epad - n_edges
    pad_ar = jnp.arange(pad, dtype=jnp.int32)
    src = jnp.concatenate([ei[1], (pad_ar * 37) % N_NODES])
    dst = jnp.concatenate([ei[0], N_NODES + pad_ar % (NPAD - N_NODES)])
    src = src.reshape(total_chunks, CHUNK)
    dst = dst.reshape(total_chunks, CHUNK)

    seg_p, cnt_p = _segment_sum_sc(features, src, dst, (cpt0, cpt1))

    feat_pad = jnp.pad(features, ((0, NPAD - N_NODES), (0, 0)))
    out = _combine_tc(feat_pad, W, seg_p, cnt_p)
    return out[:N_NODES]
